# baseline (speedup 1.0000x reference)
import functools
import math

import jax
import jax.numpy as jnp
from jax.experimental import pallas as pl
from jax.experimental.pallas import tpu as pltpu


def _hybrid_embed_kernel(ids_ref, table_hbm, pe_ref, out_ref,
                         tvmem, gvbuf, dbuf, load_sem, dsem,
                         *, scale, rows, batch, n_steps):
    c = pl.program_id(0)
    t = pl.program_id(1)

    n_load_chunks = 8
    vocab = tvmem.shape[0]
    chunk = vocab // n_load_chunks

    def load_table():
        for k in range(n_load_chunks):
            pltpu.make_async_copy(
                table_hbm.at[pl.ds(k * chunk, chunk)],
                tvmem.at[pl.ds(k * chunk, chunk), 0, :],
                load_sem,
            ).start()
        pltpu.make_async_copy(table_hbm, tvmem.at[:, 0, :], load_sem).wait()

    def issue(tile, dst_slot):
        base = tile * rows
        for r in range(rows):
            tok = ids_ref[base + r]
            pltpu.make_async_copy(
                table_hbm.at[tok],
                dbuf.at[dst_slot, r, 0],
                dsem.at[dst_slot],
            ).start()

    @pl.when(c == 0)
    def _():
        @pl.when(t == 0)
        def _():
            load_table()

        base = t * rows
        for r in range(rows):
            tok = ids_ref[base + r]
            gvbuf[r] = tvmem[tok]

        pe_big = jnp.repeat(pe_ref[...], batch, axis=0)
        out_ref[...] = gvbuf[...] * scale + pe_big

    @pl.when(c == 1)
    def _():
        slot = jax.lax.rem(t, 2)

        @pl.when(t == 0)
        def _():
            issue(n_steps, slot)

        @pl.when(t + 1 < n_steps)
        def _():
            issue(n_steps + t + 1, 1 - slot)

        pltpu.make_async_copy(dbuf.at[slot], dbuf.at[slot],
                              dsem.at[slot]).wait()

        pe_big = jnp.repeat(pe_ref[...], batch, axis=0)
        out_ref[...] = dbuf[slot] * scale + pe_big


def kernel(token_ids, word_lut, pe_table):
    seq_len, batch, nfeat = token_ids.shape
    assert nfeat == 1
    vocab, dim = word_lut.shape
    scale = float(math.sqrt(dim))

    n_cores = 2
    rows = 128
    seq_rows = rows // batch
    n_steps = seq_len * batch // rows // n_cores

    ids_flat = token_ids[:, :, 0].astype(jnp.int32).reshape(seq_len * batch)
    pe3 = pe_table[:seq_len].reshape(seq_len, 1, dim)

    body = functools.partial(
        _hybrid_embed_kernel,
        scale=scale, rows=rows, batch=batch, n_steps=n_steps,
    )

    grid_spec = pltpu.PrefetchScalarGridSpec(
        num_scalar_prefetch=1,
        grid=(n_cores, n_steps),
        in_specs=[
            pl.BlockSpec(memory_space=pl.ANY),
            pl.BlockSpec((seq_rows, 1, dim),
                         lambda c, t, ids: (c * n_steps + t, 0, 0)),
        ],
        out_specs=pl.BlockSpec((rows, 1, dim),
                               lambda c, t, ids: (c * n_steps + t, 0, 0)),
        scratch_shapes=[
            pltpu.VMEM((vocab, 1, dim), word_lut.dtype),
            pltpu.VMEM((rows, 1, dim), word_lut.dtype),
            pltpu.VMEM((2, rows, 1, dim), word_lut.dtype),
            pltpu.SemaphoreType.DMA,
            pltpu.SemaphoreType.DMA((2,)),
        ],
    )

    out = pl.pallas_call(
        body,
        grid_spec=grid_spec,
        out_shape=jax.ShapeDtypeStruct((seq_len * batch, 1, dim), word_lut.dtype),
        compiler_params=pltpu.CompilerParams(
            dimension_semantics=("parallel", "arbitrary"),
            disable_bounds_checks=True,
            vmem_limit_bytes=67108864,
        ),
    )(ids_flat, word_lut, pe3)
    return out.reshape(seq_len, batch, dim)


# speedup vs baseline: 1.9198x; 1.9198x over previous
import functools
import math

import jax
import jax.numpy as jnp
from jax.experimental import pallas as pl
from jax.experimental.pallas import tpu as pltpu


def _gather_embed_kernel(ids_ref, table_hbm, pe_ref, out_ref, gbuf, sem,
                         *, scale, tile_len, batch, n_inner):
    c = pl.program_id(0)
    j = pl.program_id(1)
    slot = jax.lax.rem(j, 2)
    rows = tile_len * batch

    def issue(tile_idx, dst_slot):
        base = tile_idx * rows
        for s in range(tile_len):
            for b in range(batch):
                tok = ids_ref[base + s * batch + b]
                pltpu.make_async_copy(
                    table_hbm.at[tok],
                    gbuf.at[dst_slot, s, b],
                    sem.at[dst_slot],
                ).start(priority=(s * batch + b) % 2)

    @pl.when(j == 0)
    def _():
        issue(c * n_inner, slot)

    @pl.when(j + 1 < n_inner)
    def _():
        issue(c * n_inner + j + 1, 1 - slot)

    pltpu.make_async_copy(gbuf.at[slot], gbuf.at[slot], sem.at[slot]).wait()

    out_ref[...] = gbuf[slot] * scale + pe_ref[...]


def kernel(token_ids, word_lut, pe_table):
    seq_len, batch, nfeat = token_ids.shape
    assert nfeat == 1
    vocab, dim = word_lut.shape
    scale = float(math.sqrt(dim))

    tile_len = 32
    n_cores = 2
    n_inner = seq_len // tile_len // n_cores

    ids_flat = token_ids[:, :, 0].reshape(seq_len * batch).astype(jnp.int32)
    pe3 = pe_table[:seq_len].reshape(seq_len, 1, dim)

    body = functools.partial(
        _gather_embed_kernel,
        scale=scale, tile_len=tile_len, batch=batch, n_inner=n_inner,
    )

    grid_spec = pltpu.PrefetchScalarGridSpec(
        num_scalar_prefetch=1,
        grid=(n_cores, n_inner),
        in_specs=[
            pl.BlockSpec(memory_space=pl.ANY),
            pl.BlockSpec((tile_len, 1, dim),
                         lambda c, j, ids: (c * n_inner + j, 0, 0)),
        ],
        out_specs=pl.BlockSpec((tile_len, batch, dim),
                               lambda c, j, ids: (c * n_inner + j, 0, 0)),
        scratch_shapes=[
            pltpu.VMEM((2, tile_len, batch, dim), word_lut.dtype),
            pltpu.SemaphoreType.DMA((2,)),
        ],
    )

    out = pl.pallas_call(
        body,
        grid_spec=grid_spec,
        out_shape=jax.ShapeDtypeStruct((seq_len, batch, dim), word_lut.dtype),
        compiler_params=pltpu.CompilerParams(
            dimension_semantics=("parallel", "arbitrary"),
            disable_bounds_checks=True,
        ),
    )(ids_flat, word_lut, pe3)
    return out
